# Initial kernel scaffold; baseline (speedup 1.0000x reference)
import jax
import jax.numpy as jnp
from jax.experimental import pallas as pl


def kernel(inputs, edge_index, W1, b1, W2, b2):
    raise NotImplementedError("write your pallas kernel here")


# speedup vs baseline: 11.0190x; 11.0190x over previous
import functools

import jax
import jax.numpy as jnp
from jax import lax
from jax.experimental import pallas as pl
from jax.experimental.pallas import tpu as pltpu
from jax.experimental.pallas import tpu_sc as plsc

N = 10000
E = 320000
D = 128

NC = 2
NS = 16
NW = NC * NS

K = 125
CHUNKS = E // NW // K
EROWS = E // K
NP = 10240
NPT = NP // NS
DEGW = 128

_MESH = plsc.VectorSubcoreMesh(core_axis_name="c", subcore_axis_name="s")


def _deg_body(edges, ones_h, zeros_h, out, idx, ones_v, acc):
    c = lax.axis_index("c")
    s = lax.axis_index("s")
    pltpu.sync_copy(zeros_h, acc.at[pl.ds(s * NPT, NPT)])
    pltpu.sync_copy(edges.at[c, pl.ds(s * (EROWS // NS), EROWS // NS)], idx)
    pltpu.sync_copy(ones_h, ones_v)
    plsc.subcore_barrier()

    @pl.loop(0, EROWS // NS)
    def _(j):
        pltpu.sync_copy(ones_v, acc.at[idx.at[j]], add=True)

    plsc.subcore_barrier()
    pltpu.sync_copy(acc.at[pl.ds(s * NPT, NPT)], out.at[c, pl.ds(s * NPT, NPT)])


_deg_kernel = functools.partial(
    pl.kernel,
    out_type=jax.ShapeDtypeStruct((2, NP, DEGW), jnp.float32),
    mesh=_MESH,
    scratch_types=[
        pltpu.VMEM((EROWS // NS, K), jnp.int32),
        pltpu.VMEM((K, DEGW), jnp.float32),
        pltpu.VMEM_SHARED((NP, DEGW), jnp.float32),
    ],
)(_deg_body)


def _agg_body(hp, srcr, dstr, zrows, out, idx_s, idx_d, rows, acc, sem):
    c = lax.axis_index("c")
    s = lax.axis_index("s")
    wid = c * NS + s
    pltpu.sync_copy(zrows, acc.at[pl.ds(s * NPT, NPT)])
    pltpu.sync_copy(srcr.at[pl.ds(wid * CHUNKS, CHUNKS)], idx_s)
    pltpu.sync_copy(dstr.at[pl.ds(wid * CHUNKS, CHUNKS)], idx_d)
    plsc.subcore_barrier()

    @pl.loop(0, CHUNKS)
    def _(j):
        pltpu.async_copy(hp.at[idx_s.at[j]], rows, sem).wait()
        pltpu.sync_copy(rows, acc.at[idx_d.at[j]], add=True)

    plsc.subcore_barrier()
    pltpu.sync_copy(acc.at[pl.ds(s * NPT, NPT)], out.at[c, pl.ds(s * NPT, NPT)])


_agg_kernel = functools.partial(
    pl.kernel,
    out_type=jax.ShapeDtypeStruct((2, NP, D), jnp.float32),
    mesh=_MESH,
    scratch_types=[
        pltpu.VMEM((CHUNKS, K), jnp.int32),
        pltpu.VMEM((CHUNKS, K), jnp.int32),
        pltpu.VMEM((K, D), jnp.float32),
        pltpu.VMEM_SHARED((NP, D), jnp.float32),
        pltpu.SemaphoreType.DMA,
    ],
)(_agg_body)


BLK = 1000
GRID = N // BLK


def _rsqrt_norm(d):
    return jnp.where(d > 0, lax.rsqrt(jnp.maximum(d, 1.0)), 0.0)


def _tc1_body(x_ref, w_ref, od_ref, o_ref):
    h = jnp.dot(x_ref[...], w_ref[...], preferred_element_type=jnp.float32,
                precision=lax.Precision.HIGHEST)
    o_ref[...] = h * _rsqrt_norm(od_ref[...])


_tc1 = pl.pallas_call(
    _tc1_body,
    out_shape=jax.ShapeDtypeStruct((N, D), jnp.float32),
    grid=(GRID,),
    in_specs=[
        pl.BlockSpec((BLK, D), lambda i: (i, 0)),
        pl.BlockSpec((D, D), lambda i: (0, 0)),
        pl.BlockSpec((BLK, 1), lambda i: (i, 0)),
    ],
    out_specs=pl.BlockSpec((BLK, D), lambda i: (i, 0)),
)


def _tc2_body(p0_ref, p1_ref, id_ref, od_ref, b_ref, w_ref, o_ref):
    nd = _rsqrt_norm(id_ref[...])
    a = (p0_ref[...] + p1_ref[...]) * nd + b_ref[...]
    h1 = jnp.maximum(a, 0.0)
    h2 = jnp.dot(h1, w_ref[...], preferred_element_type=jnp.float32,
                 precision=lax.Precision.HIGHEST)
    o_ref[...] = h2 * _rsqrt_norm(od_ref[...])


_tc2 = pl.pallas_call(
    _tc2_body,
    out_shape=jax.ShapeDtypeStruct((N, D), jnp.float32),
    grid=(GRID,),
    in_specs=[
        pl.BlockSpec((BLK, D), lambda i: (i, 0)),
        pl.BlockSpec((BLK, D), lambda i: (i, 0)),
        pl.BlockSpec((BLK, 1), lambda i: (i, 0)),
        pl.BlockSpec((BLK, 1), lambda i: (i, 0)),
        pl.BlockSpec((1, D), lambda i: (0, 0)),
        pl.BlockSpec((D, D), lambda i: (0, 0)),
    ],
    out_specs=pl.BlockSpec((BLK, D), lambda i: (i, 0)),
)


def _tc3_body(p0_ref, p1_ref, id_ref, b_ref, o_ref):
    nd = _rsqrt_norm(id_ref[...])
    z = (p0_ref[...] + p1_ref[...]) * nd + b_ref[...]
    m = jnp.max(z, axis=1, keepdims=True)
    e = jnp.exp(z - m)
    o_ref[...] = e / jnp.sum(e, axis=1, keepdims=True)


_tc3 = pl.pallas_call(
    _tc3_body,
    out_shape=jax.ShapeDtypeStruct((N, D), jnp.float32),
    grid=(GRID,),
    in_specs=[
        pl.BlockSpec((BLK, D), lambda i: (i, 0)),
        pl.BlockSpec((BLK, D), lambda i: (i, 0)),
        pl.BlockSpec((BLK, 1), lambda i: (i, 0)),
        pl.BlockSpec((1, D), lambda i: (0, 0)),
    ],
    out_specs=pl.BlockSpec((BLK, D), lambda i: (i, 0)),
)


@jax.jit
def kernel(inputs, edge_index, W1, b1, W2, b2):
    edge_index = edge_index.astype(jnp.int32)
    src2d = edge_index[0].reshape(EROWS, K)
    dst2d = edge_index[1].reshape(EROWS, K)
    edge3d = edge_index.reshape(2, EROWS, K)

    ones_h = jnp.ones((K, DEGW), jnp.float32)
    zrows = jnp.zeros((NPT, D), jnp.float32)
    zeros_h = zrows

    deg = _deg_kernel(edge3d, ones_h, zeros_h)
    outdeg = deg[0, :N, :1]
    indeg = deg[1, :N, :1]

    hp = _tc1(inputs, W1, outdeg)
    p = _agg_kernel(hp, src2d, dst2d, zrows)
    h2 = _tc2(p[0, :N], p[1, :N], indeg, outdeg, b1.reshape(1, D), W2)
    p2 = _agg_kernel(h2, src2d, dst2d, zrows)
    return _tc3(p2[0, :N], p2[1, :N], indeg, b2.reshape(1, D))


# speedup vs baseline: 13.1795x; 1.1961x over previous
import functools

import jax
import jax.numpy as jnp
from jax import lax
from jax.experimental import pallas as pl
from jax.experimental.pallas import tpu as pltpu
from jax.experimental.pallas import tpu_sc as plsc

N = 10000
E = 320000
D = 128

NC = 2
NS = 16
NW = NC * NS

K = 125
CHUNKS = E // NW // K
EROWS = E // K
NP = 10240
NPT = NP // NS
DEGW = 128

_MESH = plsc.VectorSubcoreMesh(core_axis_name="c", subcore_axis_name="s")


def _deg_body(edges, ones_h, zeros_h, out, idx, ones_v, acc):
    c = lax.axis_index("c")
    s = lax.axis_index("s")
    pltpu.sync_copy(zeros_h, acc.at[pl.ds(s * NPT, NPT)])
    pltpu.sync_copy(edges.at[c, pl.ds(s * (EROWS // NS), EROWS // NS)], idx)
    pltpu.sync_copy(ones_h, ones_v)
    plsc.subcore_barrier()

    @pl.loop(0, EROWS // NS)
    def _(j):
        pltpu.sync_copy(ones_v, acc.at[idx.at[j]], add=True)

    plsc.subcore_barrier()
    pltpu.sync_copy(acc.at[pl.ds(s * NPT, NPT)], out.at[c, pl.ds(s * NPT, NPT)])


_deg_kernel = functools.partial(
    pl.kernel,
    out_type=jax.ShapeDtypeStruct((2, NP, DEGW), jnp.float32),
    mesh=_MESH,
    scratch_types=[
        pltpu.VMEM((EROWS // NS, K), jnp.int32),
        pltpu.VMEM((K, DEGW), jnp.float32),
        pltpu.VMEM_SHARED((NP, DEGW), jnp.float32),
    ],
)(_deg_body)


def _agg_body(hp, srcr, dstr, zrows, out, ib0, ib1, idx_d, rows0, rows1, acc,
              sg0, sg1, sf0, sf1):
    c = lax.axis_index("c")
    s = lax.axis_index("s")
    wid = c * NS + s
    base = wid * CHUNKS
    pltpu.sync_copy(zrows, acc.at[pl.ds(s * NPT, NPT)])
    pltpu.sync_copy(dstr.at[pl.ds(base, CHUNKS)], idx_d)
    pltpu.sync_copy(srcr.at[pl.ds(base, 1)], ib0)
    pltpu.sync_copy(srcr.at[pl.ds(base + 1, 1)], ib1)
    plsc.subcore_barrier()

    pltpu.make_async_copy(hp.at[ib0.at[0]], rows0, sg0).start()

    @pl.loop(0, CHUNKS // 2)
    def _(jj):
        j = 2 * jj
        pltpu.make_async_copy(hp.at[ib0.at[0]], rows0, sg0).wait()
        pltpu.make_async_copy(hp.at[ib1.at[0]], rows1, sg1).start()

        @pl.when(jj < CHUNKS // 2 - 1)
        def _():
            pltpu.make_async_copy(
                srcr.at[pl.ds(base + j + 2, 1)], ib0, sf0).start()

        pltpu.sync_copy(rows0, acc.at[idx_d.at[j]], add=True)
        pltpu.make_async_copy(hp.at[ib1.at[0]], rows1, sg1).wait()

        @pl.when(jj < CHUNKS // 2 - 1)
        def _():
            pltpu.make_async_copy(
                srcr.at[pl.ds(base + j + 2, 1)], ib0, sf0).wait()
            pltpu.make_async_copy(hp.at[ib0.at[0]], rows0, sg0).start()
            pltpu.make_async_copy(
                srcr.at[pl.ds(base + j + 3, 1)], ib1, sf1).start()

        pltpu.sync_copy(rows1, acc.at[idx_d.at[j + 1]], add=True)

        @pl.when(jj < CHUNKS // 2 - 1)
        def _():
            pltpu.make_async_copy(
                srcr.at[pl.ds(base + j + 3, 1)], ib1, sf1).wait()

    plsc.subcore_barrier()
    pltpu.sync_copy(acc.at[pl.ds(s * NPT, NPT)], out.at[c, pl.ds(s * NPT, NPT)])


_agg_kernel = functools.partial(
    pl.kernel,
    out_type=jax.ShapeDtypeStruct((2, NP, D), jnp.float32),
    mesh=_MESH,
    scratch_types=[
        pltpu.VMEM((1, K), jnp.int32),
        pltpu.VMEM((1, K), jnp.int32),
        pltpu.VMEM((CHUNKS, K), jnp.int32),
        pltpu.VMEM((K, D), jnp.float32),
        pltpu.VMEM((K, D), jnp.float32),
        pltpu.VMEM_SHARED((NP, D), jnp.float32),
        pltpu.SemaphoreType.DMA,
        pltpu.SemaphoreType.DMA,
        pltpu.SemaphoreType.DMA,
        pltpu.SemaphoreType.DMA,
    ],
)(_agg_body)


BLK = 1000
GRID = N // BLK


def _rsqrt_norm(d):
    return jnp.where(d > 0, lax.rsqrt(jnp.maximum(d, 1.0)), 0.0)


def _tc1_body(x_ref, w_ref, od_ref, o_ref):
    h = jnp.dot(x_ref[...], w_ref[...], preferred_element_type=jnp.float32,
                precision=lax.Precision.HIGHEST)
    o_ref[...] = h * _rsqrt_norm(od_ref[...])


_tc1 = pl.pallas_call(
    _tc1_body,
    out_shape=jax.ShapeDtypeStruct((N, D), jnp.float32),
    grid=(GRID,),
    in_specs=[
        pl.BlockSpec((BLK, D), lambda i: (i, 0)),
        pl.BlockSpec((D, D), lambda i: (0, 0)),
        pl.BlockSpec((BLK, 1), lambda i: (i, 0)),
    ],
    out_specs=pl.BlockSpec((BLK, D), lambda i: (i, 0)),
)


def _tc2_body(p0_ref, p1_ref, id_ref, od_ref, b_ref, w_ref, o_ref):
    nd = _rsqrt_norm(id_ref[...])
    a = (p0_ref[...] + p1_ref[...]) * nd + b_ref[...]
    h1 = jnp.maximum(a, 0.0)
    h2 = jnp.dot(h1, w_ref[...], preferred_element_type=jnp.float32,
                 precision=lax.Precision.HIGHEST)
    o_ref[...] = h2 * _rsqrt_norm(od_ref[...])


_tc2 = pl.pallas_call(
    _tc2_body,
    out_shape=jax.ShapeDtypeStruct((N, D), jnp.float32),
    grid=(GRID,),
    in_specs=[
        pl.BlockSpec((BLK, D), lambda i: (i, 0)),
        pl.BlockSpec((BLK, D), lambda i: (i, 0)),
        pl.BlockSpec((BLK, 1), lambda i: (i, 0)),
        pl.BlockSpec((BLK, 1), lambda i: (i, 0)),
        pl.BlockSpec((1, D), lambda i: (0, 0)),
        pl.BlockSpec((D, D), lambda i: (0, 0)),
    ],
    out_specs=pl.BlockSpec((BLK, D), lambda i: (i, 0)),
)


def _tc3_body(p0_ref, p1_ref, id_ref, b_ref, o_ref):
    nd = _rsqrt_norm(id_ref[...])
    z = (p0_ref[...] + p1_ref[...]) * nd + b_ref[...]
    m = jnp.max(z, axis=1, keepdims=True)
    e = jnp.exp(z - m)
    o_ref[...] = e / jnp.sum(e, axis=1, keepdims=True)


_tc3 = pl.pallas_call(
    _tc3_body,
    out_shape=jax.ShapeDtypeStruct((N, D), jnp.float32),
    grid=(GRID,),
    in_specs=[
        pl.BlockSpec((BLK, D), lambda i: (i, 0)),
        pl.BlockSpec((BLK, D), lambda i: (i, 0)),
        pl.BlockSpec((BLK, 1), lambda i: (i, 0)),
        pl.BlockSpec((1, D), lambda i: (0, 0)),
    ],
    out_specs=pl.BlockSpec((BLK, D), lambda i: (i, 0)),
)


@jax.jit
def kernel(inputs, edge_index, W1, b1, W2, b2):
    edge_index = edge_index.astype(jnp.int32)
    src2d = edge_index[0].reshape(EROWS, K)
    dst2d = edge_index[1].reshape(EROWS, K)
    edge3d = edge_index.reshape(2, EROWS, K)

    ones_h = jnp.ones((K, DEGW), jnp.float32)
    zrows = jnp.zeros((NPT, D), jnp.float32)

    deg = _deg_kernel(edge3d, ones_h, zrows)
    outdeg = deg[0, :N, :1]
    indeg = deg[1, :N, :1]

    hp = _tc1(inputs, W1, outdeg)
    p = _agg_kernel(hp, src2d, dst2d, zrows)
    h2 = _tc2(p[0, :N], p[1, :N], indeg, outdeg, b1.reshape(1, D), W2)
    p2 = _agg_kernel(h2, src2d, dst2d, zrows)
    return _tc3(p2[0, :N], p2[1, :N], indeg, b2.reshape(1, D))


# speedup vs baseline: 13.7708x; 1.0449x over previous
import functools

import jax
import jax.numpy as jnp
from jax import lax
from jax.experimental import pallas as pl
from jax.experimental.pallas import tpu as pltpu
from jax.experimental.pallas import tpu_sc as plsc

N = 10000
E = 320000
D = 128

NC = 2
NS = 16
NW = NC * NS

K = 125
CHUNKS = E // NW // K
EROWS = E // K
NP = 10240
NPT = NP // NS
DEGW = 128

_MESH = plsc.VectorSubcoreMesh(core_axis_name="c", subcore_axis_name="s")


def _deg_body(edges, ones_h, zeros_h, out, idx, ones_v, acc, sd):
    c = lax.axis_index("c")
    s = lax.axis_index("s")
    pltpu.sync_copy(zeros_h, acc.at[pl.ds(s * NPT, NPT)])
    pltpu.sync_copy(edges.at[c, pl.ds(s * (EROWS // NS), EROWS // NS)], idx)
    pltpu.sync_copy(ones_h, ones_v)
    plsc.subcore_barrier()

    @pl.loop(0, EROWS // NS)
    def _(j):
        pltpu.async_copy(ones_v, acc.at[idx.at[j]], sd, add=True)

    @pl.loop(0, EROWS // NS)
    def _(j):
        pltpu.make_async_copy(ones_v, acc.at[idx.at[0]], sd).wait()

    plsc.subcore_barrier()
    pltpu.sync_copy(acc.at[pl.ds(s * NPT, NPT)], out.at[c, pl.ds(s * NPT, NPT)])


_deg_kernel = functools.partial(
    pl.kernel,
    out_type=jax.ShapeDtypeStruct((2, NP, DEGW), jnp.float32),
    mesh=_MESH,
    scratch_types=[
        pltpu.VMEM((EROWS // NS, K), jnp.int32),
        pltpu.VMEM((K, DEGW), jnp.float32),
        pltpu.VMEM_SHARED((NP, DEGW), jnp.float32),
        pltpu.SemaphoreType.DMA,
    ],
)(_deg_body)


def _agg_body(hp, srcr, dstr, zrows, out, ib0, ib1, idx_d, rows0, rows1, acc,
              sg0, sg1, sf0, sf1, ss0, ss1):
    c = lax.axis_index("c")
    s = lax.axis_index("s")
    wid = c * NS + s
    base = wid * CHUNKS
    pltpu.sync_copy(zrows, acc.at[pl.ds(s * NPT, NPT)])
    pltpu.sync_copy(dstr.at[pl.ds(base, CHUNKS)], idx_d)
    pltpu.sync_copy(srcr.at[pl.ds(base, 1)], ib0)
    pltpu.sync_copy(srcr.at[pl.ds(base + 1, 1)], ib1)
    plsc.subcore_barrier()

    pltpu.make_async_copy(hp.at[ib0.at[0]], rows0, sg0).start()
    pltpu.make_async_copy(hp.at[ib1.at[0]], rows1, sg1).start()

    @pl.loop(0, CHUNKS // 2)
    def _(jj):
        t = 2 * jj
        pltpu.make_async_copy(hp.at[ib0.at[0]], rows0, sg0).wait()

        @pl.when(jj < CHUNKS // 2 - 1)
        def _():
            pltpu.make_async_copy(
                srcr.at[pl.ds(base + t + 2, 1)], ib0, sf0).start()

        pltpu.async_copy(rows0, acc.at[idx_d.at[t]], ss0, add=True)

        @pl.when(jj > 0)
        def _():
            pltpu.make_async_copy(
                srcr.at[pl.ds(base + t + 1, 1)], ib1, sf1).wait()
            pltpu.make_async_copy(rows1, acc.at[idx_d.at[0]], ss1).wait()
            pltpu.make_async_copy(hp.at[ib1.at[0]], rows1, sg1).start()

        pltpu.make_async_copy(hp.at[ib1.at[0]], rows1, sg1).wait()

        @pl.when(jj < CHUNKS // 2 - 1)
        def _():
            pltpu.make_async_copy(
                srcr.at[pl.ds(base + t + 3, 1)], ib1, sf1).start()

        pltpu.async_copy(rows1, acc.at[idx_d.at[t + 1]], ss1, add=True)

        @pl.when(jj < CHUNKS // 2 - 1)
        def _():
            pltpu.make_async_copy(
                srcr.at[pl.ds(base + t + 2, 1)], ib0, sf0).wait()
            pltpu.make_async_copy(rows0, acc.at[idx_d.at[0]], ss0).wait()
            pltpu.make_async_copy(hp.at[ib0.at[0]], rows0, sg0).start()

    pltpu.make_async_copy(rows0, acc.at[idx_d.at[0]], ss0).wait()
    pltpu.make_async_copy(rows1, acc.at[idx_d.at[0]], ss1).wait()

    plsc.subcore_barrier()
    pltpu.sync_copy(acc.at[pl.ds(s * NPT, NPT)], out.at[c, pl.ds(s * NPT, NPT)])


_agg_kernel = functools.partial(
    pl.kernel,
    out_type=jax.ShapeDtypeStruct((2, NP, D), jnp.float32),
    mesh=_MESH,
    scratch_types=[
        pltpu.VMEM((1, K), jnp.int32),
        pltpu.VMEM((1, K), jnp.int32),
        pltpu.VMEM((CHUNKS, K), jnp.int32),
        pltpu.VMEM((K, D), jnp.float32),
        pltpu.VMEM((K, D), jnp.float32),
        pltpu.VMEM_SHARED((NP, D), jnp.float32),
        pltpu.SemaphoreType.DMA,
        pltpu.SemaphoreType.DMA,
        pltpu.SemaphoreType.DMA,
        pltpu.SemaphoreType.DMA,
        pltpu.SemaphoreType.DMA,
        pltpu.SemaphoreType.DMA,
    ],
)(_agg_body)


BLK = 1000
GRID = N // BLK


def _rsqrt_norm(d):
    return jnp.where(d > 0, lax.rsqrt(jnp.maximum(d, 1.0)), 0.0)


def _tc1_body(x_ref, w_ref, od_ref, o_ref):
    h = jnp.dot(x_ref[...], w_ref[...], preferred_element_type=jnp.float32,
                precision=lax.Precision.HIGHEST)
    o_ref[...] = h * _rsqrt_norm(od_ref[...][0, :, :1])


_tc1 = pl.pallas_call(
    _tc1_body,
    out_shape=jax.ShapeDtypeStruct((N, D), jnp.float32),
    grid=(GRID,),
    in_specs=[
        pl.BlockSpec((BLK, D), lambda i: (i, 0)),
        pl.BlockSpec((D, D), lambda i: (0, 0)),
        pl.BlockSpec((1, BLK, DEGW), lambda i: (0, i, 0)),
    ],
    out_specs=pl.BlockSpec((BLK, D), lambda i: (i, 0)),
)


def _tc2_body(p0_ref, p1_ref, id_ref, od_ref, b_ref, w_ref, o_ref):
    nd = _rsqrt_norm(id_ref[...][0, :, :1])
    a = (p0_ref[...][0] + p1_ref[...][0]) * nd + b_ref[...]
    h1 = jnp.maximum(a, 0.0)
    h2 = jnp.dot(h1, w_ref[...], preferred_element_type=jnp.float32,
                 precision=lax.Precision.HIGHEST)
    o_ref[...] = h2 * _rsqrt_norm(od_ref[...][0, :, :1])


_tc2 = pl.pallas_call(
    _tc2_body,
    out_shape=jax.ShapeDtypeStruct((N, D), jnp.float32),
    grid=(GRID,),
    in_specs=[
        pl.BlockSpec((1, BLK, D), lambda i: (0, i, 0)),
        pl.BlockSpec((1, BLK, D), lambda i: (1, i, 0)),
        pl.BlockSpec((1, BLK, DEGW), lambda i: (1, i, 0)),
        pl.BlockSpec((1, BLK, DEGW), lambda i: (0, i, 0)),
        pl.BlockSpec((1, D), lambda i: (0, 0)),
        pl.BlockSpec((D, D), lambda i: (0, 0)),
    ],
    out_specs=pl.BlockSpec((BLK, D), lambda i: (i, 0)),
)


def _tc3_body(p0_ref, p1_ref, id_ref, b_ref, o_ref):
    nd = _rsqrt_norm(id_ref[...][0, :, :1])
    z = (p0_ref[...][0] + p1_ref[...][0]) * nd + b_ref[...]
    m = jnp.max(z, axis=1, keepdims=True)
    e = jnp.exp(z - m)
    o_ref[...] = e / jnp.sum(e, axis=1, keepdims=True)


_tc3 = pl.pallas_call(
    _tc3_body,
    out_shape=jax.ShapeDtypeStruct((N, D), jnp.float32),
    grid=(GRID,),
    in_specs=[
        pl.BlockSpec((1, BLK, D), lambda i: (0, i, 0)),
        pl.BlockSpec((1, BLK, D), lambda i: (1, i, 0)),
        pl.BlockSpec((1, BLK, DEGW), lambda i: (1, i, 0)),
        pl.BlockSpec((1, D), lambda i: (0, 0)),
    ],
    out_specs=pl.BlockSpec((BLK, D), lambda i: (i, 0)),
)


@jax.jit
def kernel(inputs, edge_index, W1, b1, W2, b2):
    edge_index = edge_index.astype(jnp.int32)
    src2d = edge_index[0].reshape(EROWS, K)
    dst2d = edge_index[1].reshape(EROWS, K)
    edge3d = edge_index.reshape(2, EROWS, K)

    ones_h = jnp.ones((K, DEGW), jnp.float32)
    zrows = jnp.zeros((NPT, D), jnp.float32)

    deg = _deg_kernel(edge3d, ones_h, zrows)

    hp = _tc1(inputs, W1, deg)
    p = _agg_kernel(hp, src2d, dst2d, zrows)
    h2 = _tc2(p, p, deg, deg, b1.reshape(1, D), W2)
    p2 = _agg_kernel(h2, src2d, dst2d, zrows)
    return _tc3(p2, p2, deg, b2.reshape(1, D))


# speedup vs baseline: 13.8994x; 1.0093x over previous
import functools

import jax
import jax.numpy as jnp
from jax import lax
from jax.experimental import pallas as pl
from jax.experimental.pallas import tpu as pltpu
from jax.experimental.pallas import tpu_sc as plsc

N = 10000
E = 320000
D = 128

NC = 2
NS = 16
NW = NC * NS

K = 125
CHUNKS = E // NW // K
EROWS = E // K
NP = 10240
NPT = NP // NS
DEGW = 128

_MESH = plsc.VectorSubcoreMesh(core_axis_name="c", subcore_axis_name="s")


def _deg_body(edges, ones_h, zeros_h, out, idx, ones_v, acc, sd):
    c = lax.axis_index("c")
    s = lax.axis_index("s")
    pltpu.sync_copy(zeros_h, acc.at[pl.ds(s * NPT, NPT)])
    pltpu.sync_copy(edges.at[c, pl.ds(s * (EROWS // NS), EROWS // NS)], idx)
    pltpu.sync_copy(ones_h, ones_v)
    plsc.subcore_barrier()

    @pl.loop(0, EROWS // NS)
    def _(j):
        pltpu.async_copy(ones_v, acc.at[idx.at[j]], sd, add=True)

    @pl.loop(0, EROWS // NS)
    def _(j):
        pltpu.make_async_copy(ones_v, acc.at[idx.at[0]], sd).wait()

    plsc.subcore_barrier()
    pltpu.sync_copy(acc.at[pl.ds(s * NPT, NPT)], out.at[c, pl.ds(s * NPT, NPT)])


_deg_kernel = functools.partial(
    pl.kernel,
    out_type=jax.ShapeDtypeStruct((2, NP, DEGW), jnp.float32),
    mesh=_MESH,
    scratch_types=[
        pltpu.VMEM((EROWS // NS, K), jnp.int32),
        pltpu.VMEM((K, DEGW), jnp.float32),
        pltpu.VMEM_SHARED((NP, DEGW), jnp.float32),
        pltpu.SemaphoreType.DMA,
    ],
)(_deg_body)


def _agg_body(hp, srcr, dstr, zrows, out, ib0, ib1, idx_d, rows0, rows1, acc,
              sg0, sg1, sf0, sf1, ss0, ss1):
    c = lax.axis_index("c")
    s = lax.axis_index("s")
    wid = c * NS + s
    base = wid * CHUNKS
    pltpu.sync_copy(zrows, acc.at[pl.ds(s * NPT, NPT)])
    pltpu.sync_copy(dstr.at[pl.ds(base, CHUNKS)], idx_d)
    pltpu.sync_copy(srcr.at[pl.ds(base, 1)], ib0)
    pltpu.sync_copy(srcr.at[pl.ds(base + 1, 1)], ib1)
    plsc.subcore_barrier()

    pltpu.make_async_copy(hp.at[ib0.at[0]], rows0, sg0).start()
    pltpu.make_async_copy(hp.at[ib1.at[0]], rows1, sg1).start()

    @pl.loop(0, CHUNKS // 2)
    def _(jj):
        t = 2 * jj
        pltpu.make_async_copy(hp.at[ib0.at[0]], rows0, sg0).wait()

        @pl.when(jj < CHUNKS // 2 - 1)
        def _():
            pltpu.make_async_copy(
                srcr.at[pl.ds(base + t + 2, 1)], ib0, sf0).start()

        pltpu.async_copy(rows0, acc.at[idx_d.at[t]], ss0, add=True)

        @pl.when(jj > 0)
        def _():
            pltpu.make_async_copy(
                srcr.at[pl.ds(base + t + 1, 1)], ib1, sf1).wait()
            pltpu.make_async_copy(rows1, acc.at[idx_d.at[0]], ss1).wait()
            pltpu.make_async_copy(hp.at[ib1.at[0]], rows1, sg1).start()

        pltpu.make_async_copy(hp.at[ib1.at[0]], rows1, sg1).wait()

        @pl.when(jj < CHUNKS // 2 - 1)
        def _():
            pltpu.make_async_copy(
                srcr.at[pl.ds(base + t + 3, 1)], ib1, sf1).start()

        pltpu.async_copy(rows1, acc.at[idx_d.at[t + 1]], ss1, add=True)

        @pl.when(jj < CHUNKS // 2 - 1)
        def _():
            pltpu.make_async_copy(
                srcr.at[pl.ds(base + t + 2, 1)], ib0, sf0).wait()
            pltpu.make_async_copy(rows0, acc.at[idx_d.at[0]], ss0).wait()
            pltpu.make_async_copy(hp.at[ib0.at[0]], rows0, sg0).start()

    pltpu.make_async_copy(rows0, acc.at[idx_d.at[0]], ss0).wait()
    pltpu.make_async_copy(rows1, acc.at[idx_d.at[0]], ss1).wait()

    plsc.subcore_barrier()
    pltpu.sync_copy(acc.at[pl.ds(s * NPT, NPT)], out.at[c, pl.ds(s * NPT, NPT)])


_agg_kernel = functools.partial(
    pl.kernel,
    out_type=jax.ShapeDtypeStruct((2, NP, D), jnp.float32),
    mesh=_MESH,
    scratch_types=[
        pltpu.VMEM((1, K), jnp.int32),
        pltpu.VMEM((1, K), jnp.int32),
        pltpu.VMEM((CHUNKS, K), jnp.int32),
        pltpu.VMEM((K, D), jnp.float32),
        pltpu.VMEM((K, D), jnp.float32),
        pltpu.VMEM_SHARED((NP, D), jnp.float32),
        pltpu.SemaphoreType.DMA,
        pltpu.SemaphoreType.DMA,
        pltpu.SemaphoreType.DMA,
        pltpu.SemaphoreType.DMA,
        pltpu.SemaphoreType.DMA,
        pltpu.SemaphoreType.DMA,
    ],
)(_agg_body)


BLK = 1000
GRID = N // BLK


def _rsqrt_norm(d):
    return jnp.where(d > 0, lax.rsqrt(jnp.maximum(d, 1.0)), 0.0)


def _tc1a_body(x_ref, w_ref, o_ref):
    o_ref[...] = jnp.dot(x_ref[...], w_ref[...],
                         preferred_element_type=jnp.float32,
                         precision=lax.Precision.HIGHEST)


_tc1a = pl.pallas_call(
    _tc1a_body,
    out_shape=jax.ShapeDtypeStruct((N, D), jnp.float32),
    grid=(GRID,),
    in_specs=[
        pl.BlockSpec((BLK, D), lambda i: (i, 0)),
        pl.BlockSpec((D, D), lambda i: (0, 0)),
    ],
    out_specs=pl.BlockSpec((BLK, D), lambda i: (i, 0)),
)


def _tc1b_body(h_ref, od_ref, o_ref):
    o_ref[...] = h_ref[...] * _rsqrt_norm(od_ref[...][0, :, :1])


_tc1b = pl.pallas_call(
    _tc1b_body,
    out_shape=jax.ShapeDtypeStruct((N, D), jnp.float32),
    grid=(GRID,),
    in_specs=[
        pl.BlockSpec((BLK, D), lambda i: (i, 0)),
        pl.BlockSpec((1, BLK, DEGW), lambda i: (0, i, 0)),
    ],
    out_specs=pl.BlockSpec((BLK, D), lambda i: (i, 0)),
)


def _tc2_body(p0_ref, p1_ref, id_ref, od_ref, b_ref, w_ref, o_ref):
    nd = _rsqrt_norm(id_ref[...][0, :, :1])
    a = (p0_ref[...][0] + p1_ref[...][0]) * nd + b_ref[...]
    h1 = jnp.maximum(a, 0.0)
    h2 = jnp.dot(h1, w_ref[...], preferred_element_type=jnp.float32,
                 precision=lax.Precision.HIGHEST)
    o_ref[...] = h2 * _rsqrt_norm(od_ref[...][0, :, :1])


_tc2 = pl.pallas_call(
    _tc2_body,
    out_shape=jax.ShapeDtypeStruct((N, D), jnp.float32),
    grid=(GRID,),
    in_specs=[
        pl.BlockSpec((1, BLK, D), lambda i: (0, i, 0)),
        pl.BlockSpec((1, BLK, D), lambda i: (1, i, 0)),
        pl.BlockSpec((1, BLK, DEGW), lambda i: (1, i, 0)),
        pl.BlockSpec((1, BLK, DEGW), lambda i: (0, i, 0)),
        pl.BlockSpec((1, D), lambda i: (0, 0)),
        pl.BlockSpec((D, D), lambda i: (0, 0)),
    ],
    out_specs=pl.BlockSpec((BLK, D), lambda i: (i, 0)),
)


def _tc3_body(p0_ref, p1_ref, id_ref, b_ref, o_ref):
    nd = _rsqrt_norm(id_ref[...][0, :, :1])
    z = (p0_ref[...][0] + p1_ref[...][0]) * nd + b_ref[...]
    m = jnp.max(z, axis=1, keepdims=True)
    e = jnp.exp(z - m)
    o_ref[...] = e / jnp.sum(e, axis=1, keepdims=True)


_tc3 = pl.pallas_call(
    _tc3_body,
    out_shape=jax.ShapeDtypeStruct((N, D), jnp.float32),
    grid=(GRID,),
    in_specs=[
        pl.BlockSpec((1, BLK, D), lambda i: (0, i, 0)),
        pl.BlockSpec((1, BLK, D), lambda i: (1, i, 0)),
        pl.BlockSpec((1, BLK, DEGW), lambda i: (1, i, 0)),
        pl.BlockSpec((1, D), lambda i: (0, 0)),
    ],
    out_specs=pl.BlockSpec((BLK, D), lambda i: (i, 0)),
)


@jax.jit
def kernel(inputs, edge_index, W1, b1, W2, b2):
    edge_index = edge_index.astype(jnp.int32)
    src2d = edge_index[0].reshape(EROWS, K)
    dst2d = edge_index[1].reshape(EROWS, K)
    edge3d = edge_index.reshape(2, EROWS, K)

    ones_h = jnp.ones((K, DEGW), jnp.float32)
    zrows = jnp.zeros((NPT, D), jnp.float32)

    deg = _deg_kernel(edge3d, ones_h, zrows)

    h = _tc1a(inputs, W1)
    hp = _tc1b(h, deg)
    p = _agg_kernel(hp, src2d, dst2d, zrows)
    h2 = _tc2(p, p, deg, deg, b1.reshape(1, D), W2)
    p2 = _agg_kernel(h2, src2d, dst2d, zrows)
    return _tc3(p2, p2, deg, b2.reshape(1, D))


# speedup vs baseline: 13.9217x; 1.0016x over previous
import functools

import jax
import jax.numpy as jnp
from jax import lax
from jax.experimental import pallas as pl
from jax.experimental.pallas import tpu as pltpu
from jax.experimental.pallas import tpu_sc as plsc

N = 10000
E = 320000
D = 128

NC = 2
NS = 16
NW = NC * NS

K = 125
CHUNKS = E // NW // K
EROWS = E // K
NP = 10240
NPT = NP // NS
DEGW = 128

_MESH = plsc.VectorSubcoreMesh(core_axis_name="c", subcore_axis_name="s")


def _deg_body(edges, ones_h, zeros_h, out, idx, ones_v, acc, sd):
    c = lax.axis_index("c")
    s = lax.axis_index("s")
    pltpu.sync_copy(zeros_h, acc.at[pl.ds(s * NPT, NPT)])
    pltpu.sync_copy(edges.at[c, pl.ds(s * (EROWS // NS), EROWS // NS)], idx)
    pltpu.sync_copy(ones_h, ones_v)
    plsc.subcore_barrier()

    @pl.loop(0, EROWS // NS)
    def _(j):
        pltpu.async_copy(ones_v, acc.at[idx.at[j]], sd, add=True)

    @pl.loop(0, EROWS // NS)
    def _(j):
        pltpu.make_async_copy(ones_v, acc.at[idx.at[0]], sd).wait()

    plsc.subcore_barrier()
    pltpu.sync_copy(acc.at[pl.ds(s * NPT, NPT)], out.at[c, pl.ds(s * NPT, NPT)])


_deg_kernel = functools.partial(
    pl.kernel,
    out_type=jax.ShapeDtypeStruct((2, NP, DEGW), jnp.float32),
    mesh=_MESH,
    scratch_types=[
        pltpu.VMEM((EROWS // NS, K), jnp.int32),
        pltpu.VMEM((K, DEGW), jnp.float32),
        pltpu.VMEM_SHARED((NP, DEGW), jnp.float32),
        pltpu.SemaphoreType.DMA,
    ],
)(_deg_body)


def _agg_body(hp, srcr, dstr, zrows, out, ib0, ib1, idx_d, rows0, rows1, acc,
              sg0, sg1, sf0, sf1, ss0, ss1):
    c = lax.axis_index("c")
    s = lax.axis_index("s")
    wid = c * NS + s
    base = wid * CHUNKS
    pltpu.sync_copy(zrows, acc.at[pl.ds(s * NPT, NPT)])
    pltpu.sync_copy(dstr.at[pl.ds(base, CHUNKS)], idx_d)
    pltpu.sync_copy(srcr.at[pl.ds(base, 1)], ib0)
    pltpu.sync_copy(srcr.at[pl.ds(base + 1, 1)], ib1)
    plsc.subcore_barrier()

    pltpu.make_async_copy(hp.at[ib0.at[0]], rows0, sg0).start()
    pltpu.make_async_copy(hp.at[ib1.at[0]], rows1, sg1).start()

    @pl.loop(0, CHUNKS // 2)
    def _(jj):
        t = 2 * jj
        pltpu.make_async_copy(hp.at[ib0.at[0]], rows0, sg0).wait()

        @pl.when(jj < CHUNKS // 2 - 1)
        def _():
            pltpu.make_async_copy(
                srcr.at[pl.ds(base + t + 2, 1)], ib0, sf0).start()

        pltpu.async_copy(rows0, acc.at[idx_d.at[t]], ss0, add=True)

        @pl.when(jj > 0)
        def _():
            pltpu.make_async_copy(
                srcr.at[pl.ds(base + t + 1, 1)], ib1, sf1).wait()
            pltpu.make_async_copy(rows1, acc.at[idx_d.at[0]], ss1).wait()
            pltpu.make_async_copy(hp.at[ib1.at[0]], rows1, sg1).start()

        pltpu.make_async_copy(hp.at[ib1.at[0]], rows1, sg1).wait()

        @pl.when(jj < CHUNKS // 2 - 1)
        def _():
            pltpu.make_async_copy(
                srcr.at[pl.ds(base + t + 3, 1)], ib1, sf1).start()

        pltpu.async_copy(rows1, acc.at[idx_d.at[t + 1]], ss1, add=True)

        @pl.when(jj < CHUNKS // 2 - 1)
        def _():
            pltpu.make_async_copy(
                srcr.at[pl.ds(base + t + 2, 1)], ib0, sf0).wait()
            pltpu.make_async_copy(rows0, acc.at[idx_d.at[0]], ss0).wait()
            pltpu.make_async_copy(hp.at[ib0.at[0]], rows0, sg0).start()

    pltpu.make_async_copy(rows0, acc.at[idx_d.at[0]], ss0).wait()
    pltpu.make_async_copy(rows1, acc.at[idx_d.at[0]], ss1).wait()

    plsc.subcore_barrier()
    pltpu.sync_copy(acc.at[pl.ds(s * NPT, NPT)], out.at[c, pl.ds(s * NPT, NPT)])


_agg_kernel = functools.partial(
    pl.kernel,
    out_type=jax.ShapeDtypeStruct((2, NP, D), jnp.float32),
    mesh=_MESH,
    scratch_types=[
        pltpu.VMEM((1, K), jnp.int32),
        pltpu.VMEM((1, K), jnp.int32),
        pltpu.VMEM((CHUNKS, K), jnp.int32),
        pltpu.VMEM((K, D), jnp.float32),
        pltpu.VMEM((K, D), jnp.float32),
        pltpu.VMEM_SHARED((NP, D), jnp.float32),
        pltpu.SemaphoreType.DMA,
        pltpu.SemaphoreType.DMA,
        pltpu.SemaphoreType.DMA,
        pltpu.SemaphoreType.DMA,
        pltpu.SemaphoreType.DMA,
        pltpu.SemaphoreType.DMA,
    ],
)(_agg_body)


BLK = 1000
GRID = N // BLK


def _rsqrt_norm(d):
    return jnp.where(d > 0, lax.rsqrt(jnp.maximum(d, 1.0)), 0.0)


def _tc1a_body(x_ref, w_ref, o_ref):
    o_ref[...] = jnp.dot(x_ref[...], w_ref[...],
                         preferred_element_type=jnp.float32,
                         precision=lax.Precision.HIGHEST)


_tc1a = pl.pallas_call(
    _tc1a_body,
    out_shape=jax.ShapeDtypeStruct((N, D), jnp.float32),
    grid=(GRID,),
    in_specs=[
        pl.BlockSpec((BLK, D), lambda i: (i, 0)),
        pl.BlockSpec((D, D), lambda i: (0, 0)),
    ],
    out_specs=pl.BlockSpec((BLK, D), lambda i: (i, 0)),
)


def _tc1b_body(h_ref, od_ref, o_ref):
    o_ref[...] = h_ref[...] * _rsqrt_norm(od_ref[...][0, :, :1])


_tc1b = pl.pallas_call(
    _tc1b_body,
    out_shape=jax.ShapeDtypeStruct((N, D), jnp.float32),
    grid=(GRID,),
    in_specs=[
        pl.BlockSpec((BLK, D), lambda i: (i, 0)),
        pl.BlockSpec((1, BLK, DEGW), lambda i: (0, i, 0)),
    ],
    out_specs=pl.BlockSpec((BLK, D), lambda i: (i, 0)),
)


def _tc2_body(p0_ref, p1_ref, deg_ref, b_ref, w_ref, o_ref):
    d = deg_ref[...]
    nd = _rsqrt_norm(d[1, :, :1])
    a = (p0_ref[...][0] + p1_ref[...][0]) * nd + b_ref[...]
    h1 = jnp.maximum(a, 0.0)
    h2 = jnp.dot(h1, w_ref[...], preferred_element_type=jnp.float32,
                 precision=lax.Precision.HIGHEST)
    o_ref[...] = h2 * _rsqrt_norm(d[0, :, :1])


_tc2 = pl.pallas_call(
    _tc2_body,
    out_shape=jax.ShapeDtypeStruct((N, D), jnp.float32),
    grid=(GRID,),
    in_specs=[
        pl.BlockSpec((1, BLK, D), lambda i: (0, i, 0)),
        pl.BlockSpec((1, BLK, D), lambda i: (1, i, 0)),
        pl.BlockSpec((2, BLK, DEGW), lambda i: (0, i, 0)),
        pl.BlockSpec((1, D), lambda i: (0, 0)),
        pl.BlockSpec((D, D), lambda i: (0, 0)),
    ],
    out_specs=pl.BlockSpec((BLK, D), lambda i: (i, 0)),
)


def _tc3_body(p0_ref, p1_ref, id_ref, b_ref, o_ref):
    nd = _rsqrt_norm(id_ref[...][0, :, :1])
    z = (p0_ref[...][0] + p1_ref[...][0]) * nd + b_ref[...]
    m = jnp.max(z, axis=1, keepdims=True)
    e = jnp.exp(z - m)
    o_ref[...] = e / jnp.sum(e, axis=1, keepdims=True)


_tc3 = pl.pallas_call(
    _tc3_body,
    out_shape=jax.ShapeDtypeStruct((N, D), jnp.float32),
    grid=(GRID,),
    in_specs=[
        pl.BlockSpec((1, BLK, D), lambda i: (0, i, 0)),
        pl.BlockSpec((1, BLK, D), lambda i: (1, i, 0)),
        pl.BlockSpec((1, BLK, DEGW), lambda i: (1, i, 0)),
        pl.BlockSpec((1, D), lambda i: (0, 0)),
    ],
    out_specs=pl.BlockSpec((BLK, D), lambda i: (i, 0)),
)


@jax.jit
def kernel(inputs, edge_index, W1, b1, W2, b2):
    edge_index = edge_index.astype(jnp.int32)
    src2d = edge_index[0].reshape(EROWS, K)
    dst2d = edge_index[1].reshape(EROWS, K)
    edge3d = edge_index.reshape(2, EROWS, K)

    ones_h = jnp.ones((K, DEGW), jnp.float32)
    zrows = jnp.zeros((NPT, D), jnp.float32)

    deg = _deg_kernel(edge3d, ones_h, zrows)

    h = _tc1a(inputs, W1)
    hp = _tc1b(h, deg)
    p = _agg_kernel(hp, src2d, dst2d, zrows)
    h2 = _tc2(p, p, deg, b1.reshape(1, D), W2)
    p2 = _agg_kernel(h2, src2d, dst2d, zrows)
    return _tc3(p2, p2, deg, b2.reshape(1, D))
